# baseline (speedup 1.0000x reference)
import jax
import jax.numpy as jnp
from jax import lax
from jax.experimental import pallas as pl
from jax.experimental.pallas import tpu as pltpu
from jax.experimental.pallas import tpu_sc as plsc

_B, _C, _H, _W = 64, 768, 24, 24
_NC, _NS = 2, 16
_NW = _NC * _NS
_BATCH_PER_W = _B // _NW


def _sc_body(x_hbm, out_hbm, sem):
    wid = lax.axis_index("s") * _NC + lax.axis_index("c")
    b0 = wid * _BATCH_PER_W

    def issue(c, b):
        pltpu.make_async_copy(
            x_hbm.at[b, _C - 1 - c], out_hbm.at[b, c], sem
        ).start()
        return b

    for j in range(_BATCH_PER_W):
        lax.fori_loop(0, _C, issue, b0 + j)

    pltpu.make_async_copy(
        x_hbm.at[pl.ds(b0, _BATCH_PER_W)],
        out_hbm.at[pl.ds(b0, _BATCH_PER_W)],
        sem,
    ).wait()


def kernel(input):
    run = pl.kernel(
        _sc_body,
        out_type=jax.ShapeDtypeStruct((_B, _C, _H, _W), jnp.float32),
        mesh=plsc.VectorSubcoreMesh(core_axis_name="c", subcore_axis_name="s"),
        scratch_types=[
            pltpu.SemaphoreType.DMA,
        ],
    )
    return run(input)


# speedup vs baseline: 36.4829x; 36.4829x over previous
import jax
import jax.numpy as jnp
from jax import lax
from jax.experimental import pallas as pl
from jax.experimental.pallas import tpu as pltpu
from jax.experimental.pallas import tpu_sc as plsc

_B, _C, _H, _W = 64, 768, 24, 24
_D = _H * _W
_NC, _NS = 2, 16
_NW = _NC * _NS
_BATCH_PER_W = _B // _NW
_K = 64
_NCHUNK = _C // _K
_NV = _D // 16


def _sc_body(x_hbm, out_hbm, buf_a, buf_b, sem_in, sem_out):
    wid = lax.axis_index("s") * _NC + lax.axis_index("c")
    b0 = wid * _BATCH_PER_W

    def chunk(i, b):
        c0 = i * _K
        src0 = _C - c0 - _K
        pltpu.async_copy(x_hbm.at[b, pl.ds(src0, _K)], buf_a, sem_in).wait()

        def rev_row(r, carry):
            s = _K - 1 - r
            for v in range(_NV):
                buf_b[r, pl.ds(v * 16, 16)] = buf_a[s, pl.ds(v * 16, 16)]
            return carry

        lax.fori_loop(0, _K, rev_row, 0)
        pltpu.async_copy(buf_b, out_hbm.at[b, pl.ds(c0, _K)], sem_out).wait()
        return b

    for j in range(_BATCH_PER_W):
        lax.fori_loop(0, _NCHUNK, chunk, b0 + j)


def kernel(input):
    x = input.reshape(_B, _C, _D)
    run = pl.kernel(
        _sc_body,
        out_type=jax.ShapeDtypeStruct((_B, _C, _D), jnp.float32),
        mesh=plsc.VectorSubcoreMesh(core_axis_name="c", subcore_axis_name="s"),
        scratch_types=[
            pltpu.VMEM((_K, _D), jnp.float32),
            pltpu.VMEM((_K, _D), jnp.float32),
            pltpu.SemaphoreType.DMA,
            pltpu.SemaphoreType.DMA,
        ],
    )
    out = run(x)
    return out.reshape(_B, _C, _H, _W)


# speedup vs baseline: 52.0364x; 1.4263x over previous
import jax
import jax.numpy as jnp
from jax import lax
from jax.experimental import pallas as pl
from jax.experimental.pallas import tpu as pltpu
from jax.experimental.pallas import tpu_sc as plsc

_B, _C, _H, _W = 64, 768, 24, 24
_D = _H * _W
_NC, _NS = 2, 16
_NW = _NC * _NS
_BATCH_PER_W = _B // _NW
_K = 64
_NCHUNK = _C // _K
_NPAIR = _NCHUNK // 2
_NV = _D // 16


def _sc_body(x_hbm, out_hbm, buf0, buf1, si0, si1, so0, so1):
    wid = lax.axis_index("s") * _NC + lax.axis_index("c")
    b0 = wid * _BATCH_PER_W

    def make_rev(buf):
        def rev(r, carry):
            s = _K - 1 - r
            for v in range(_NV):
                sl = pl.ds(v * 16, 16)
                a = buf[r, sl]
                b = buf[s, sl]
                buf[r, sl] = b
                buf[s, sl] = a
            return carry
        return rev

    rev0, rev1 = make_rev(buf0), make_rev(buf1)

    for j in range(_BATCH_PER_W):
        b = b0 + j

        def src(c):
            return x_hbm.at[b, pl.ds(_C - c - _K, _K)]

        def dst(c):
            return out_hbm.at[b, pl.ds(c, _K)]

        pltpu.async_copy(src(0), buf0, si0)

        def pair(p, carry):
            c0 = 2 * p * _K
            c1 = c0 + _K
            pltpu.make_async_copy(src(c0), buf0, si0).wait()

            @pl.when(p >= 1)
            def _():
                pltpu.make_async_copy(buf1, dst(c0 - _K), so1).wait()

            pltpu.async_copy(src(c1), buf1, si1)
            lax.fori_loop(0, _K // 2, rev0, 0)
            pltpu.async_copy(buf0, dst(c0), so0)
            pltpu.make_async_copy(src(c1), buf1, si1).wait()

            @pl.when(p <= _NPAIR - 2)
            def _():
                pltpu.make_async_copy(buf0, dst(c0), so0).wait()
                pltpu.async_copy(src(c1 + _K), buf0, si0)

            lax.fori_loop(0, _K // 2, rev1, 0)
            pltpu.async_copy(buf1, dst(c1), so1)
            return carry

        lax.fori_loop(0, _NPAIR, pair, 0)
        pltpu.make_async_copy(buf0, dst(_C - 2 * _K), so0).wait()
        pltpu.make_async_copy(buf1, dst(_C - _K), so1).wait()


def kernel(input):
    x = input.reshape(_B, _C, _D)
    run = pl.kernel(
        _sc_body,
        out_type=jax.ShapeDtypeStruct((_B, _C, _D), jnp.float32),
        mesh=plsc.VectorSubcoreMesh(core_axis_name="c", subcore_axis_name="s"),
        scratch_types=[
            pltpu.VMEM((_K, _D), jnp.float32),
            pltpu.VMEM((_K, _D), jnp.float32),
            pltpu.SemaphoreType.DMA,
            pltpu.SemaphoreType.DMA,
            pltpu.SemaphoreType.DMA,
            pltpu.SemaphoreType.DMA,
        ],
    )
    out = run(x)
    return out.reshape(_B, _C, _H, _W)


# speedup vs baseline: 59.9882x; 1.1528x over previous
import jax
import jax.numpy as jnp
from jax import lax
from jax.experimental import pallas as pl
from jax.experimental.pallas import tpu as pltpu
from jax.experimental.pallas import tpu_sc as plsc

_B, _C, _H, _W = 64, 768, 24, 24
_D = _H * _W
_NC, _NS = 2, 16
_NW = _NC * _NS
_BATCH_PER_W = _B // _NW
_K = 64
_NCHUNK = _C // _K
_NPAIR = _NCHUNK // 2


def _sc_body(x_hbm, out_hbm, buf0, buf1, si0, si1, so0, so1):
    wid = lax.axis_index("s") * _NC + lax.axis_index("c")
    b0 = wid * _BATCH_PER_W

    for j in range(_BATCH_PER_W):
        b = b0 + j

        def issue_rows(c0, buf, sem):
            def one(r, carry):
                pltpu.async_copy(x_hbm.at[b, _C - 1 - c0 - r], buf.at[r], sem)
                return carry
            lax.fori_loop(0, _K, one, 0)

        def drain(buf, sem):
            pltpu.make_async_copy(x_hbm.at[b, pl.ds(0, _K)], buf, sem).wait()

        def pair(p, carry):
            c0 = 2 * p * _K
            c1 = c0 + _K

            @pl.when(p >= 1)
            def _():
                pltpu.make_async_copy(buf0, out_hbm.at[b, pl.ds(c0 - 2 * _K, _K)], so0).wait()

            issue_rows(c0, buf0, si0)

            @pl.when(p >= 1)
            def _():
                pltpu.make_async_copy(buf1, out_hbm.at[b, pl.ds(c0 - _K, _K)], so1).wait()

            issue_rows(c1, buf1, si1)
            drain(buf0, si0)
            pltpu.async_copy(buf0, out_hbm.at[b, pl.ds(c0, _K)], so0)
            drain(buf1, si1)
            pltpu.async_copy(buf1, out_hbm.at[b, pl.ds(c1, _K)], so1)
            return carry

        lax.fori_loop(0, _NPAIR, pair, 0)
        pltpu.make_async_copy(buf0, out_hbm.at[b, pl.ds(_C - 2 * _K, _K)], so0).wait()
        pltpu.make_async_copy(buf1, out_hbm.at[b, pl.ds(_C - _K, _K)], so1).wait()


def kernel(input):
    x = input.reshape(_B, _C, _D)
    run = pl.kernel(
        _sc_body,
        out_type=jax.ShapeDtypeStruct((_B, _C, _D), jnp.float32),
        mesh=plsc.VectorSubcoreMesh(core_axis_name="c", subcore_axis_name="s"),
        scratch_types=[
            pltpu.VMEM((_K, _D), jnp.float32),
            pltpu.VMEM((_K, _D), jnp.float32),
            pltpu.SemaphoreType.DMA,
            pltpu.SemaphoreType.DMA,
            pltpu.SemaphoreType.DMA,
            pltpu.SemaphoreType.DMA,
        ],
    )
    out = run(x)
    return out.reshape(_B, _C, _H, _W)


# speedup vs baseline: 60.0197x; 1.0005x over previous
import jax
import jax.numpy as jnp
from jax import lax
from jax.experimental import pallas as pl
from jax.experimental.pallas import tpu as pltpu
from jax.experimental.pallas import tpu_sc as plsc

_B, _C, _H, _W = 64, 768, 24, 24
_D = _H * _W
_NC, _NS = 2, 16
_NW = _NC * _NS
_BATCH_PER_W = _B // _NW
_K = 64
_NCHUNK = _C // _K
_NB = 3
_NTRIPLE = _NCHUNK // _NB


def _sc_body(x_hbm, out_hbm, buf0, buf1, buf2, si0, si1, si2, so0, so1, so2):
    wid = lax.axis_index("s") * _NC + lax.axis_index("c")
    b0 = wid * _BATCH_PER_W
    bufs = (buf0, buf1, buf2)
    isems = (si0, si1, si2)
    osems = (so0, so1, so2)

    for j in range(_BATCH_PER_W):
        b = b0 + j

        def issue_rows(c0, buf, sem):
            def one(r, carry):
                pltpu.async_copy(x_hbm.at[b, _C - 1 - c0 - r], buf.at[r], sem)
                return carry
            lax.fori_loop(0, _K, one, 0)

        def triple(p, carry):
            base = _NB * p * _K
            for q in range(_NB):
                c = base + q * _K

                @pl.when(p >= 1)
                def _():
                    pltpu.make_async_copy(
                        bufs[q], out_hbm.at[b, pl.ds(c - _NB * _K, _K)],
                        osems[q],
                    ).wait()

                issue_rows(c, bufs[q], isems[q])
            for q in range(_NB):
                c = base + q * _K
                pltpu.make_async_copy(
                    x_hbm.at[b, pl.ds(0, _K)], bufs[q], isems[q]
                ).wait()
                pltpu.async_copy(bufs[q], out_hbm.at[b, pl.ds(c, _K)], osems[q])
            return carry

        lax.fori_loop(0, _NTRIPLE, triple, 0)
        for q in range(_NB):
            c = (_NCHUNK - _NB + q) * _K
            pltpu.make_async_copy(
                bufs[q], out_hbm.at[b, pl.ds(c, _K)], osems[q]
            ).wait()


def kernel(input):
    x = input.reshape(_B, _C, _D)
    run = pl.kernel(
        _sc_body,
        out_type=jax.ShapeDtypeStruct((_B, _C, _D), jnp.float32),
        mesh=plsc.VectorSubcoreMesh(core_axis_name="c", subcore_axis_name="s"),
        scratch_types=[
            pltpu.VMEM((_K, _D), jnp.float32),
            pltpu.VMEM((_K, _D), jnp.float32),
            pltpu.VMEM((_K, _D), jnp.float32),
            pltpu.SemaphoreType.DMA,
            pltpu.SemaphoreType.DMA,
            pltpu.SemaphoreType.DMA,
            pltpu.SemaphoreType.DMA,
            pltpu.SemaphoreType.DMA,
            pltpu.SemaphoreType.DMA,
        ],
    )
    out = run(x)
    return out.reshape(_B, _C, _H, _W)


# speedup vs baseline: 188.1223x; 3.1343x over previous
import jax
import jax.numpy as jnp
from jax import lax
from jax.experimental import pallas as pl
from jax.experimental.pallas import tpu as pltpu
from jax.experimental.pallas import tpu_sc as plsc

_B, _C, _H, _W = 64, 768, 24, 24
_NC, _NS = 2, 16
_NW = _NC * _NS
_BATCH_PER_W = _B // _NW
_HCHUNK = 2
_NCHUNK_B = _H // _HCHUNK
_NT = _BATCH_PER_W * _NCHUNK_B
_NB = 3
_NTRIPLE = _NT // _NB
_NVPAIR = _C // 32


def _sc_body(y_hbm, out_hbm, buf0, buf1, buf2, si0, si1, si2, so0, so1, so2):
    wid = lax.axis_index("s") * _NC + lax.axis_index("c")
    b0 = wid * _BATCH_PER_W
    bufs = (buf0, buf1, buf2)
    isems = (si0, si1, si2)
    osems = (so0, so1, so2)

    def chunk_slice(ref, t):
        b = b0 + t // _NCHUNK_B
        h0 = (t % _NCHUNK_B) * _HCHUNK
        return ref.at[b, pl.ds(h0, _HCHUNK)]

    def rev_rows(buf):
        def row(r, carry):
            i = r // _W
            k = r % _W
            for v in range(_NVPAIR):
                lo = pl.ds(16 * v, 16)
                hi = pl.ds(_C - 16 * (v + 1), 16)
                a = buf[i, k, lo]
                z = buf[i, k, hi]
                buf[i, k, lo] = lax.rev(z, (0,))
                buf[i, k, hi] = lax.rev(a, (0,))
            return carry

        lax.fori_loop(0, _HCHUNK * _W, row, 0)

    def triple(p, carry):
        for q in range(_NB):
            t = _NB * p + q

            @pl.when(p >= 1)
            def _():
                pltpu.make_async_copy(
                    bufs[q], chunk_slice(out_hbm, t - _NB), osems[q]
                ).wait()

            pltpu.async_copy(chunk_slice(y_hbm, t), bufs[q], isems[q])
        for q in range(_NB):
            t = _NB * p + q
            pltpu.make_async_copy(
                chunk_slice(y_hbm, t), bufs[q], isems[q]
            ).wait()
            rev_rows(bufs[q])
            pltpu.async_copy(bufs[q], chunk_slice(out_hbm, t), osems[q])
        return carry

    lax.fori_loop(0, _NTRIPLE, triple, 0)
    for q in range(_NB):
        t = _NT - _NB + q
        pltpu.make_async_copy(
            bufs[q], chunk_slice(out_hbm, t), osems[q]
        ).wait()


def kernel(input):
    y = input.transpose(0, 2, 3, 1)
    run = pl.kernel(
        _sc_body,
        out_type=jax.ShapeDtypeStruct((_B, _H, _W, _C), jnp.float32),
        mesh=plsc.VectorSubcoreMesh(core_axis_name="c", subcore_axis_name="s"),
        scratch_types=[
            pltpu.VMEM((_HCHUNK, _W, _C), jnp.float32),
            pltpu.VMEM((_HCHUNK, _W, _C), jnp.float32),
            pltpu.VMEM((_HCHUNK, _W, _C), jnp.float32),
            pltpu.SemaphoreType.DMA,
            pltpu.SemaphoreType.DMA,
            pltpu.SemaphoreType.DMA,
            pltpu.SemaphoreType.DMA,
            pltpu.SemaphoreType.DMA,
            pltpu.SemaphoreType.DMA,
        ],
    )
    out = run(y)
    return out.transpose(0, 3, 1, 2)
